# Initial kernel scaffold; baseline (speedup 1.0000x reference)
import jax
import jax.numpy as jnp
from jax.experimental import pallas as pl


def kernel(minute, hour, day, week, month, E_minute, E_hour, E_day, E_week, E_month, W, b):
    raise NotImplementedError("write your pallas kernel here")


# speedup vs baseline: 9.8726x; 9.8726x over previous
import functools

import jax
import jax.numpy as jnp
import numpy as np
from jax import lax
from jax.experimental import pallas as pl
from jax.experimental.pallas import tpu as pltpu
from jax.experimental.pallas import tpu_sc as plsc

B, S = 4096, 200
D = 64
N_TOK = B * S
NW = 32
TPW = N_TOK // NW
C = 128
NCH = TPW // C
T1_ROWS = 60 * 24
T2_ROWS = 31 * 7 * 12
T2_PAD = 2608


def _onehot(idx, n):
    return (idx[:, None] == np.arange(n)[None, :]).astype(np.float32)


_r1 = np.arange(T1_ROWS)
_G1M = _onehot(_r1 // 24, 60)
_G1H = _onehot(_r1 % 24, 24)
_r2 = np.arange(T2_PAD)
_G2D = _onehot(np.minimum(_r2 // 84, 30), 31)
_G2W = _onehot((_r2 // 12) % 7, 7)
_G2MO = _onehot(_r2 % 12, 12)


def _tables_body(em, eh, ed, ew, emo, w, bias, g1m, g1h, g2d, g2w, g2mo,
                 t1_ref, t2_ref):
    f32 = jnp.float32
    am = jnp.dot(em[...], w[0:12, :], preferred_element_type=f32)
    ah = jnp.dot(eh[...], w[12:24, :], preferred_element_type=f32)
    ad = jnp.dot(ed[...], w[24:36, :], preferred_element_type=f32)
    aw = jnp.dot(ew[...], w[36:48, :], preferred_element_type=f32)
    amo = jnp.dot(emo[...], w[48:60, :], preferred_element_type=f32)
    t1_ref[...] = (jnp.dot(g1m[...], am, preferred_element_type=f32)
                   + jnp.dot(g1h[...], ah, preferred_element_type=f32)
                   + bias[...])
    t2_ref[...] = (jnp.dot(g2d[...], ad, preferred_element_type=f32)
                   + jnp.dot(g2w[...], aw, preferred_element_type=f32)
                   + jnp.dot(g2mo[...], amo, preferred_element_type=f32))


def _build_tables(E_minute, E_hour, E_day, E_week, E_month, W, b):
    return pl.pallas_call(
        _tables_body,
        out_shape=(
            jax.ShapeDtypeStruct((T1_ROWS, D), jnp.float32),
            jax.ShapeDtypeStruct((T2_PAD, D), jnp.float32),
        ),
    )(E_minute, E_hour, E_day, E_week, E_month, W, b.reshape(1, D),
      _G1M, _G1H, _G2D, _G2W, _G2MO)


def _sc_body(m_hbm, h_hbm, d_hbm, w_hbm, mo_hbm, t1_hbm, t2_hbm, out_hbm,
             rawv, i1v, i2v, outv, r2v, sem1, sem2):
    wid = lax.axis_index("s") * 2 + lax.axis_index("c")
    wbase = wid * TPW

    def chunk(ch, _):
        base = wbase + ch * C
        pltpu.sync_copy(m_hbm.at[pl.ds(base, C)], rawv.at[0])
        pltpu.sync_copy(h_hbm.at[pl.ds(base, C)], rawv.at[1])
        pltpu.sync_copy(d_hbm.at[pl.ds(base, C)], rawv.at[2])
        pltpu.sync_copy(w_hbm.at[pl.ds(base, C)], rawv.at[3])
        pltpu.sync_copy(mo_hbm.at[pl.ds(base, C)], rawv.at[4])
        for g in range(C // 16):
            sl = pl.ds(g * 16, 16)
            i1v[sl] = rawv[0, sl] * 24 + rawv[1, sl]
            i2v[sl] = rawv[2, sl] * 84 + rawv[3, sl] * 12 + rawv[4, sl]
        cp1 = pltpu.async_copy(t1_hbm.at[i1v], outv, sem1)
        cp2 = pltpu.async_copy(t2_hbm.at[i2v], r2v, sem2)
        cp1.wait()
        cp2.wait()

        def acc(t, _):
            for q in range(D // 16):
                sl = pl.ds(q * 16, 16)
                plsc.addupdate(outv.at[t, sl], r2v[t, sl])
            return 0

        lax.fori_loop(0, C, acc, 0)
        pltpu.sync_copy(outv, out_hbm.at[pl.ds(base, C)])
        return 0

    lax.fori_loop(0, NCH, chunk, 0)


@jax.jit
def _run(minute, hour, day, week, month,
         E_minute, E_hour, E_day, E_week, E_month, W, b):
    t1, t2 = _build_tables(E_minute, E_hour, E_day, E_week, E_month, W, b)
    mesh = plsc.VectorSubcoreMesh(core_axis_name="c", subcore_axis_name="s")
    sc = pl.kernel(
        _sc_body,
        out_type=jax.ShapeDtypeStruct((N_TOK, D), jnp.float32),
        mesh=mesh,
        compiler_params=pltpu.CompilerParams(use_tc_tiling_on_sc=False),
        scratch_types=[
            pltpu.VMEM((5, C), jnp.int32),
            pltpu.VMEM((C,), jnp.int32),
            pltpu.VMEM((C,), jnp.int32),
            pltpu.VMEM((C, D), jnp.float32),
            pltpu.VMEM((C, D), jnp.float32),
            pltpu.SemaphoreType.DMA,
            pltpu.SemaphoreType.DMA,
        ],
    )
    out = sc(minute.reshape(-1), hour.reshape(-1), day.reshape(-1),
             week.reshape(-1), month.reshape(-1), t1, t2)
    return out.reshape(B, S, D)


def kernel(minute, hour, day, week, month,
           E_minute, E_hour, E_day, E_week, E_month, W, b):
    return _run(minute, hour, day, week, month,
                E_minute, E_hour, E_day, E_week, E_month, W, b)


# speedup vs baseline: 14.7861x; 1.4977x over previous
import functools

import jax
import jax.numpy as jnp
import numpy as np
from jax import lax
from jax.experimental import pallas as pl
from jax.experimental.pallas import tpu as pltpu
from jax.experimental.pallas import tpu_sc as plsc

B, S = 4096, 200
D = 64
N_TOK = B * S
NW = 32
TPW = N_TOK // NW
C = 128
NCH = TPW // C
T1_ROWS = 60 * 24
T2_ROWS = 31 * 7 * 12
T2_PAD = 2608


def _onehot(idx, n):
    return (idx[:, None] == np.arange(n)[None, :]).astype(np.float32)


_r1 = np.arange(T1_ROWS)
_G1M = _onehot(_r1 // 24, 60)
_G1H = _onehot(_r1 % 24, 24)
_r2 = np.arange(T2_PAD)
_G2D = _onehot(np.minimum(_r2 // 84, 30), 31)
_G2W = _onehot((_r2 // 12) % 7, 7)
_G2MO = _onehot(_r2 % 12, 12)


def _tables_body(em, eh, ed, ew, emo, w, bias, g1m, g1h, g2d, g2w, g2mo,
                 t1_ref, t2_ref):
    f32 = jnp.float32
    am = jnp.dot(em[...], w[0:12, :], preferred_element_type=f32)
    ah = jnp.dot(eh[...], w[12:24, :], preferred_element_type=f32)
    ad = jnp.dot(ed[...], w[24:36, :], preferred_element_type=f32)
    aw = jnp.dot(ew[...], w[36:48, :], preferred_element_type=f32)
    amo = jnp.dot(emo[...], w[48:60, :], preferred_element_type=f32)
    t1_ref[...] = (jnp.dot(g1m[...], am, preferred_element_type=f32)
                   + jnp.dot(g1h[...], ah, preferred_element_type=f32)
                   + bias[...])
    t2_ref[...] = (jnp.dot(g2d[...], ad, preferred_element_type=f32)
                   + jnp.dot(g2w[...], aw, preferred_element_type=f32)
                   + jnp.dot(g2mo[...], amo, preferred_element_type=f32))


def _build_tables(E_minute, E_hour, E_day, E_week, E_month, W, b):
    return pl.pallas_call(
        _tables_body,
        out_shape=(
            jax.ShapeDtypeStruct((T1_ROWS, D), jnp.float32),
            jax.ShapeDtypeStruct((T2_PAD, D), jnp.float32),
        ),
    )(E_minute, E_hour, E_day, E_week, E_month, W, b.reshape(1, D),
      _G1M, _G1H, _G2D, _G2W, _G2MO)


def _sc_body(idx_hbm, t1_hbm, t2_hbm, out_hbm,
             rawv, i1v, i2v, outv, r2v,
             si0, si1, sg0, sg1, sh0, sh1, sw0, sw1):
    wid = lax.axis_index("s") * 2 + lax.axis_index("c")
    wbase = wid * TPW
    sidx = (si0, si1)
    sg_t1 = (sg0, sg1)
    sg_t2 = (sh0, sh1)
    swb = (sw0, sw1)

    def issue_idx(g, slot):
        base = wbase + g * C
        pltpu.make_async_copy(idx_hbm.at[:, pl.ds(base, C)],
                              rawv.at[slot], sidx[slot]).start()

    def wait_idx(slot):
        pltpu.make_async_copy(idx_hbm.at[:, pl.ds(0, C)],
                              rawv.at[slot], sidx[slot]).wait()

    def combine(slot):
        for g in range(C // 16):
            sl = pl.ds(g * 16, 16)
            i1v[slot, sl] = rawv[slot, 0, sl] * 24 + rawv[slot, 1, sl]
            i2v[slot, sl] = (rawv[slot, 2, sl] * 84 + rawv[slot, 3, sl] * 12
                             + rawv[slot, 4, sl])

    def issue_gathers(slot):
        pltpu.make_async_copy(t1_hbm.at[i1v.at[slot]], outv.at[slot],
                              sg_t1[slot]).start()
        pltpu.make_async_copy(t2_hbm.at[i2v.at[slot]], r2v.at[slot],
                              sg_t2[slot]).start()

    def wait_gathers(slot):
        pltpu.make_async_copy(t1_hbm.at[pl.ds(0, C)], outv.at[slot],
                              sg_t1[slot]).wait()
        pltpu.make_async_copy(t2_hbm.at[pl.ds(0, C)], r2v.at[slot],
                              sg_t2[slot]).wait()

    def accumulate(slot):
        def acc(t, _):
            for tt in range(8):
                for q in range(D // 16):
                    sl = pl.ds(q * 16, 16)
                    plsc.addupdate(outv.at[slot, t * 8 + tt, sl],
                                   r2v[slot, t * 8 + tt, sl])
            return 0
        lax.fori_loop(0, C // 8, acc, 0, unroll=2)

    def issue_wb(g, slot):
        base = wbase + g * C
        pltpu.make_async_copy(outv.at[slot], out_hbm.at[pl.ds(base, C)],
                              swb[slot]).start()

    def wait_wb(slot):
        pltpu.make_async_copy(outv.at[slot], out_hbm.at[pl.ds(0, C)],
                              swb[slot]).wait()

    def body(g, buf, first=False, last_pair=False, no_idx=False):
        nbuf = 1 - buf
        if not last_pair:
            wait_idx(nbuf)
            combine(nbuf)
            if not first:
                wait_wb(nbuf)
            issue_gathers(nbuf)
            if not no_idx:
                issue_idx(g + 2, buf)
        else:
            wait_wb(nbuf)
        wait_gathers(buf)
        accumulate(buf)
        issue_wb(g, buf)

    issue_idx(0, 0)
    wait_idx(0)
    combine(0)
    issue_gathers(0)
    issue_idx(1, 1)
    body(0, 0, first=True)

    def pair(p, _):
        g = 2 * p + 1
        body(g, 1)
        body(g + 1, 0)
        return 0

    lax.fori_loop(0, (NCH - 3) // 2, pair, 0)
    body(NCH - 3, 1)
    body(NCH - 2, 0, no_idx=True)
    body(NCH - 1, 1, last_pair=True)
    wait_wb(1)


@jax.jit
def _run(minute, hour, day, week, month,
         E_minute, E_hour, E_day, E_week, E_month, W, b):
    t1, t2 = _build_tables(E_minute, E_hour, E_day, E_week, E_month, W, b)
    idx = jnp.stack([minute.reshape(-1), hour.reshape(-1), day.reshape(-1),
                     week.reshape(-1), month.reshape(-1)], axis=0)
    mesh = plsc.VectorSubcoreMesh(core_axis_name="c", subcore_axis_name="s")
    sc = pl.kernel(
        _sc_body,
        out_type=jax.ShapeDtypeStruct((N_TOK, D), jnp.float32),
        mesh=mesh,
        compiler_params=pltpu.CompilerParams(use_tc_tiling_on_sc=False),
        scratch_types=[
            pltpu.VMEM((2, 5, C), jnp.int32),
            pltpu.VMEM((2, C), jnp.int32),
            pltpu.VMEM((2, C), jnp.int32),
            pltpu.VMEM((2, C, D), jnp.float32),
            pltpu.VMEM((2, C, D), jnp.float32),
        ] + [pltpu.SemaphoreType.DMA] * 8,
    )
    out = sc(idx, t1, t2)
    return out.reshape(B, S, D)


def kernel(minute, hour, day, week, month,
           E_minute, E_hour, E_day, E_week, E_month, W, b):
    return _run(minute, hour, day, week, month,
                E_minute, E_hour, E_day, E_week, E_month, W, b)
